# baseline (speedup 1.0000x reference)
import functools

import jax
import jax.numpy as jnp
from jax import lax
from jax.experimental import pallas as pl
from jax.experimental.pallas import tpu as pltpu
from jax.experimental.pallas import tpu_sc as plsc

_C = 16
_TEX = 1024
_NPIX = 4 * 512 * 512
_NW = 32
_PT = _NPIX // _NW
_N = 512
_NCH = _PT // _N
_G = _N // 16
_IPW = 512 * 512
_CLIP_LO = -123.68
_CLIP_HI = 151.061


def _clip_transpose(texf):
    bw = 2048

    def body(t_ref, o_ref):
        o_ref[...] = jnp.clip(t_ref[...], _CLIP_LO, _CLIP_HI).T

    return pl.pallas_call(
        body,
        grid=(_TEX * _TEX // bw,),
        in_specs=[pl.BlockSpec((_C, bw), lambda i: (0, i))],
        out_specs=pl.BlockSpec((bw, _C), lambda i: (i, 0)),
        out_shape=jax.ShapeDtypeStruct((_TEX * _TEX, _C), jnp.float32),
    )(texf)


def _sc_sample(xf, table):
    mesh = plsc.VectorSubcoreMesh(core_axis_name="c", subcore_axis_name="s")

    @functools.partial(
        pl.kernel,
        mesh=mesh,
        out_type=jax.ShapeDtypeStruct((4, _C, _IPW), jnp.float32),
        compiler_params=pltpu.CompilerParams(
            needs_layout_passes=False, use_tc_tiling_on_sc=False),
        scratch_types=[
            pltpu.VMEM((2 * _N,), jnp.float32),
            pltpu.VMEM((_N // 128, 128), jnp.int32),
            pltpu.VMEM((_N // 128, 128), jnp.int32),
            pltpu.VMEM((_N // 128, 128), jnp.int32),
            pltpu.VMEM((_N // 128, 128), jnp.int32),
            pltpu.VMEM((_N,), jnp.float32),
            pltpu.VMEM((_N,), jnp.float32),
            pltpu.VMEM((_N, _C), jnp.float32),
            pltpu.VMEM((_N, _C), jnp.float32),
            pltpu.VMEM((_N, _C), jnp.float32),
            pltpu.VMEM((_N, _C), jnp.float32),
            pltpu.VMEM((_C * _N,), jnp.float32),
            pltpu.SemaphoreType.DMA,
        ],
    )
    def k(x_hbm, tab_hbm, out_hbm, xbuf, i00, i01, i10, i11, wxb, wyb,
          r00, r01, r10, r11, obuf, gsem):
        wid = lax.axis_index("s") * 2 + lax.axis_index("c")
        tile_base = wid * _PT
        b = wid // (_IPW // _PT)
        boff = (wid % (_IPW // _PT)) * _PT

        def chunk_body(ci, carry):
            base = tile_base + ci * _N
            pltpu.sync_copy(x_hbm.at[pl.ds(2 * base, 2 * _N)], xbuf)

            def idx_body(g, c2):
                pv2 = (g * 16 + lax.iota(jnp.int32, 16)) * 2
                gx = plsc.load_gather(xbuf, [pv2])
                gy = plsc.load_gather(xbuf, [pv2 + 1])
                ix = jnp.clip((gx + 1.0) * 0.5 * (_TEX - 1), 0.0, _TEX - 1.0)
                iy = jnp.clip((gy + 1.0) * 0.5 * (_TEX - 1), 0.0, _TEX - 1.0)
                x0 = ix.astype(jnp.int32)
                y0 = iy.astype(jnp.int32)
                wx = ix - x0.astype(jnp.float32)
                wy = iy - y0.astype(jnp.float32)
                x1 = jnp.minimum(x0 + 1, _TEX - 1)
                y1 = jnp.minimum(y0 + 1, _TEX - 1)
                row0 = y0 * _TEX
                row1 = y1 * _TEX
                j = g // 8
                sl = pl.ds((g % 8) * 16, 16)
                i00[j, sl] = row0 + x0
                i01[j, sl] = row0 + x1
                i10[j, sl] = row1 + x0
                i11[j, sl] = row1 + x1
                fl = pl.ds(g * 16, 16)
                wxb[fl] = wx
                wyb[fl] = wy
                return c2

            lax.fori_loop(0, _G, idx_body, 0)

            descs = []
            for j in range(_N // 128):
                rs = pl.ds(j * 128, 128)
                for ib, rb in ((i00, r00), (i01, r01), (i10, r10), (i11, r11)):
                    descs.append(
                        pltpu.async_copy(tab_hbm.at[ib.at[j]], rb.at[rs], gsem))
            for dsc in descs:
                dsc.wait()

            ch_base = lax.iota(jnp.int32, 16) * _N

            def blend_body(g, c2):
                for i in range(16):
                    p = g * 16 + i
                    wx = plsc.load_gather(wxb, [jnp.zeros((16,), jnp.int32) + p])
                    wy = plsc.load_gather(wyb, [jnp.zeros((16,), jnp.int32) + p])
                    v00 = r00[p, :]
                    v01 = r01[p, :]
                    v10 = r10[p, :]
                    v11 = r11[p, :]
                    top = v00 + wx * (v01 - v00)
                    bot = v10 + wx * (v11 - v10)
                    res = top + wy * (bot - top)
                    plsc.store_scatter(obuf, [ch_base + p], res)
                return c2

            lax.fori_loop(0, _G, blend_body, 0)

            off = boff + ci * _N
            for c in range(_C):
                pltpu.sync_copy(obuf.at[pl.ds(c * _N, _N)],
                                out_hbm.at[b, c, pl.ds(off, _N)])
            return carry

        lax.fori_loop(0, _NCH, chunk_body, 0)

    return k(xf, table)


def kernel(x, texture):
    xf = x.reshape(2 * _NPIX)
    texf = texture.reshape(_C, _TEX * _TEX)
    table = _clip_transpose(texf)
    outf = _sc_sample(xf, table)
    return outf.reshape(4, _C, 512, 512)


# speedup vs baseline: 1.3059x; 1.3059x over previous
import functools

import jax
import jax.numpy as jnp
from jax import lax
from jax.experimental import pallas as pl
from jax.experimental.pallas import tpu as pltpu
from jax.experimental.pallas import tpu_sc as plsc

_C = 16
_TEX = 1024
_NPIX = 4 * 512 * 512
_NW = 32
_PT = _NPIX // _NW
_N = 512
_NCH = _PT // _N
_G = _N // 16
_IPW = 512 * 512
_CLIP_LO = -123.68
_CLIP_HI = 151.061


def _clip_transpose(tex_tiles):
    mesh = plsc.VectorSubcoreMesh(core_axis_name="c", subcore_axis_name="s")
    blocks_per_w = (128 * 8) // _NW

    @functools.partial(
        pl.kernel,
        mesh=mesh,
        out_type=jax.ShapeDtypeStruct((_TEX, _TEX * _C), jnp.float32),
        compiler_params=pltpu.CompilerParams(
            needs_layout_passes=False, use_tc_tiling_on_sc=False),
        scratch_types=[
            pltpu.VMEM((_C, 1024), jnp.float32),
            pltpu.VMEM((1024 * _C,), jnp.float32),
            pltpu.SemaphoreType.DMA,
            pltpu.SemaphoreType.DMA,
        ],
    )
    def k(tex_hbm, tab_hbm, tbuf, obuf, isem, osem):
        wid = lax.axis_index("s") * 2 + lax.axis_index("c")

        def block_body(bi, carry):
            blk = wid * blocks_per_w + bi
            yt = blk // 8
            xt = blk % 8
            ind = [
                pltpu.async_copy(tex_hbm.at[c, yt, xt], tbuf.at[c], isem)
                for c in range(_C)
            ]
            for d in ind:
                d.wait()

            def tr_body(g, c2):
                t16 = g * 16 + lax.iota(jnp.int32, 16)
                for c in range(_C):
                    v = jnp.clip(tbuf[c, pl.ds(g * 16, 16)], _CLIP_LO, _CLIP_HI)
                    plsc.store_scatter(obuf, [t16 * _C + c], v)
                return c2

            lax.fori_loop(0, 64, tr_body, 0)

            outd = [
                pltpu.async_copy(
                    obuf.at[pl.ds(yin * 128 * _C, 128 * _C)],
                    tab_hbm.at[yt * 8 + yin, pl.ds(xt * 128 * _C, 128 * _C)],
                    osem)
                for yin in range(8)
            ]
            for d in outd:
                d.wait()
            return carry

        lax.fori_loop(0, blocks_per_w, block_body, 0)

    return k(tex_tiles)


def _sc_sample(xf, table):
    mesh = plsc.VectorSubcoreMesh(core_axis_name="c", subcore_axis_name="s")

    @functools.partial(
        pl.kernel,
        mesh=mesh,
        out_type=jax.ShapeDtypeStruct((4, _C, _IPW), jnp.float32),
        compiler_params=pltpu.CompilerParams(
            needs_layout_passes=False, use_tc_tiling_on_sc=False),
        scratch_types=[
            pltpu.VMEM((2 * _N,), jnp.float32),
            pltpu.VMEM((_N // 128, 128), jnp.int32),
            pltpu.VMEM((_N // 128, 128), jnp.int32),
            pltpu.VMEM((_N // 128, 128), jnp.int32),
            pltpu.VMEM((_N // 128, 128), jnp.int32),
            pltpu.VMEM((_N,), jnp.float32),
            pltpu.VMEM((_N,), jnp.float32),
            pltpu.VMEM((_N, _C), jnp.float32),
            pltpu.VMEM((_N, _C), jnp.float32),
            pltpu.VMEM((_N, _C), jnp.float32),
            pltpu.VMEM((_N, _C), jnp.float32),
            pltpu.VMEM((_C * _N,), jnp.float32),
            pltpu.SemaphoreType.DMA,
        ],
    )
    def k(x_hbm, tab_hbm, out_hbm, xbuf, i00, i01, i10, i11, wxb, wyb,
          r00, r01, r10, r11, obuf, gsem):
        wid = lax.axis_index("s") * 2 + lax.axis_index("c")
        tile_base = wid * _PT
        b = wid // (_IPW // _PT)
        boff = (wid % (_IPW // _PT)) * _PT

        def chunk_body(ci, carry):
            base = tile_base + ci * _N
            pltpu.sync_copy(x_hbm.at[pl.ds(2 * base, 2 * _N)], xbuf)

            def idx_body(g, c2):
                pv2 = (g * 16 + lax.iota(jnp.int32, 16)) * 2
                gx = plsc.load_gather(xbuf, [pv2])
                gy = plsc.load_gather(xbuf, [pv2 + 1])
                ix = jnp.clip((gx + 1.0) * 0.5 * (_TEX - 1), 0.0, _TEX - 1.0)
                iy = jnp.clip((gy + 1.0) * 0.5 * (_TEX - 1), 0.0, _TEX - 1.0)
                x0 = ix.astype(jnp.int32)
                y0 = iy.astype(jnp.int32)
                wx = ix - x0.astype(jnp.float32)
                wy = iy - y0.astype(jnp.float32)
                x1 = jnp.minimum(x0 + 1, _TEX - 1)
                y1 = jnp.minimum(y0 + 1, _TEX - 1)
                row0 = y0 * _TEX
                row1 = y1 * _TEX
                j = g // 8
                sl = pl.ds((g % 8) * 16, 16)
                i00[j, sl] = row0 + x0
                i01[j, sl] = row0 + x1
                i10[j, sl] = row1 + x0
                i11[j, sl] = row1 + x1
                fl = pl.ds(g * 16, 16)
                wxb[fl] = wx
                wyb[fl] = wy
                return c2

            lax.fori_loop(0, _G, idx_body, 0)

            descs = []
            for j in range(_N // 128):
                rs = pl.ds(j * 128, 128)
                for ib, rb in ((i00, r00), (i01, r01), (i10, r10), (i11, r11)):
                    descs.append(
                        pltpu.async_copy(tab_hbm.at[ib.at[j]], rb.at[rs], gsem))
            for dsc in descs:
                dsc.wait()

            ch_base = lax.iota(jnp.int32, 16) * _N

            def blend_body(g, c2):
                for i in range(16):
                    p = g * 16 + i
                    wx = plsc.load_gather(wxb, [jnp.zeros((16,), jnp.int32) + p])
                    wy = plsc.load_gather(wyb, [jnp.zeros((16,), jnp.int32) + p])
                    v00 = r00[p, :]
                    v01 = r01[p, :]
                    v10 = r10[p, :]
                    v11 = r11[p, :]
                    top = v00 + wx * (v01 - v00)
                    bot = v10 + wx * (v11 - v10)
                    res = top + wy * (bot - top)
                    plsc.store_scatter(obuf, [ch_base + p], res)
                return c2

            lax.fori_loop(0, _G, blend_body, 0)

            off = boff + ci * _N
            for c in range(_C):
                pltpu.sync_copy(obuf.at[pl.ds(c * _N, _N)],
                                out_hbm.at[b, c, pl.ds(off, _N)])
            return carry

        lax.fori_loop(0, _NCH, chunk_body, 0)

    return k(xf, table)


def kernel(x, texture):
    xf = x.reshape(2 * _NPIX)
    tex_tiles = (texture.reshape(_C, 128, 8, 8, 128)
                 .transpose(0, 1, 3, 2, 4)
                 .reshape(_C, 128, 8, 1024))
    table = _clip_transpose(tex_tiles).reshape(_TEX * _TEX, _C)
    outf = _sc_sample(xf, table)
    return outf.reshape(4, _C, 512, 512)


# speedup vs baseline: 1.3984x; 1.0709x over previous
import functools

import jax
import jax.numpy as jnp
from jax import lax
from jax.experimental import pallas as pl
from jax.experimental.pallas import tpu as pltpu
from jax.experimental.pallas import tpu_sc as plsc

_C = 16
_TEX = 1024
_NPIX = 4 * 512 * 512
_NW = 32
_PT = _NPIX // _NW
_N = 512
_NCH = _PT // _N
_G = _N // 16
_IPW = 512 * 512
_CLIP_LO = -123.68
_CLIP_HI = 151.061


def _clip_transpose(tex_tiles):
    mesh = plsc.VectorSubcoreMesh(core_axis_name="c", subcore_axis_name="s")
    blocks_per_w = (128 * 8) // _NW

    @functools.partial(
        pl.kernel,
        mesh=mesh,
        out_type=jax.ShapeDtypeStruct((_TEX, _TEX * _C), jnp.float32),
        compiler_params=pltpu.CompilerParams(
            needs_layout_passes=False, use_tc_tiling_on_sc=False),
        scratch_types=[
            pltpu.VMEM((_C, 1024), jnp.float32),
            pltpu.VMEM((1024 * _C,), jnp.float32),
            pltpu.SemaphoreType.DMA,
            pltpu.SemaphoreType.DMA,
        ],
    )
    def k(tex_hbm, tab_hbm, tbuf, obuf, isem, osem):
        wid = lax.axis_index("s") * 2 + lax.axis_index("c")

        def block_body(bi, carry):
            blk = wid * blocks_per_w + bi
            yt = blk // 8
            xt = blk % 8
            ind = [
                pltpu.async_copy(tex_hbm.at[c, yt, xt], tbuf.at[c], isem)
                for c in range(_C)
            ]
            for d in ind:
                d.wait()

            def tr_body(g, c2):
                t16 = g * 16 + lax.iota(jnp.int32, 16)
                for c in range(_C):
                    v = jnp.clip(tbuf[c, pl.ds(g * 16, 16)], _CLIP_LO, _CLIP_HI)
                    plsc.store_scatter(obuf, [t16 * _C + c], v)
                return c2

            lax.fori_loop(0, 64, tr_body, 0)

            outd = [
                pltpu.async_copy(
                    obuf.at[pl.ds(yin * 128 * _C, 128 * _C)],
                    tab_hbm.at[yt * 8 + yin, pl.ds(xt * 128 * _C, 128 * _C)],
                    osem)
                for yin in range(8)
            ]
            for d in outd:
                d.wait()
            return carry

        lax.fori_loop(0, blocks_per_w, block_body, 0)

    return k(tex_tiles)


_SP = 1024
_SG = _SP // 16
_SPW = (_NPIX // _SP) // _NW


def _sc_sample(xf, table):
    mesh = plsc.VectorSubcoreMesh(core_axis_name="c", subcore_axis_name="s")

    @functools.partial(
        pl.kernel,
        mesh=mesh,
        out_type=jax.ShapeDtypeStruct((4, _C, 64, 4, _SP), jnp.float32),
        compiler_params=pltpu.CompilerParams(
            needs_layout_passes=False, use_tc_tiling_on_sc=False),
        scratch_types=[
            pltpu.VMEM((2 * _SP,), jnp.float32),
            pltpu.VMEM((_SP // 128, 128), jnp.int32),
            pltpu.VMEM((_SP // 128, 128), jnp.int32),
            pltpu.VMEM((_SP // 128, 128), jnp.int32),
            pltpu.VMEM((_SP // 128, 128), jnp.int32),
            pltpu.VMEM((_SP,), jnp.float32),
            pltpu.VMEM((_SP,), jnp.float32),
            pltpu.VMEM((_SP, _C), jnp.float32),
            pltpu.VMEM((_SP, _C), jnp.float32),
            pltpu.VMEM((_SP, _C), jnp.float32),
            pltpu.VMEM((_SP, _C), jnp.float32),
            pltpu.VMEM((_C * _SP,), jnp.float32),
            pltpu.SemaphoreType.DMA,
            pltpu.SemaphoreType.DMA,
        ],
    )
    def k(x_hbm, tab_hbm, out_hbm, xbuf, i00, i01, i10, i11, wxb, wyb,
          r00, r01, r10, r11, obuf, gsem, osem):
        wid = lax.axis_index("s") * 2 + lax.axis_index("c")

        def strip_body(si, carry):
            s = wid * _SPW + si
            b = s // 256
            rem = s % 256
            yt = rem // 4
            xt = rem % 4

            xd = [
                pltpu.async_copy(
                    x_hbm.at[pl.ds(
                        2 * (b * _IPW + (yt * 8 + yin) * 512 + xt * 128), 256)],
                    xbuf.at[pl.ds(yin * 256, 256)], gsem)
                for yin in range(8)
            ]
            for d in xd:
                d.wait()

            def idx_body(g, c2):
                pv2 = (g * 16 + lax.iota(jnp.int32, 16)) * 2
                gx = plsc.load_gather(xbuf, [pv2])
                gy = plsc.load_gather(xbuf, [pv2 + 1])
                ix = jnp.clip((gx + 1.0) * 0.5 * (_TEX - 1), 0.0, _TEX - 1.0)
                iy = jnp.clip((gy + 1.0) * 0.5 * (_TEX - 1), 0.0, _TEX - 1.0)
                x0 = ix.astype(jnp.int32)
                y0 = iy.astype(jnp.int32)
                wx = ix - x0.astype(jnp.float32)
                wy = iy - y0.astype(jnp.float32)
                x1 = jnp.minimum(x0 + 1, _TEX - 1)
                y1 = jnp.minimum(y0 + 1, _TEX - 1)
                row0 = y0 * _TEX
                row1 = y1 * _TEX
                j = g // 8
                sl = pl.ds((g % 8) * 16, 16)
                i00[j, sl] = row0 + x0
                i01[j, sl] = row0 + x1
                i10[j, sl] = row1 + x0
                i11[j, sl] = row1 + x1
                fl = pl.ds(g * 16, 16)
                wxb[fl] = wx
                wyb[fl] = wy
                return c2

            lax.fori_loop(0, _SG, idx_body, 0)

            descs = []
            for j in range(_SP // 128):
                rs = pl.ds(j * 128, 128)
                for ib, rb in ((i00, r00), (i01, r01), (i10, r10), (i11, r11)):
                    descs.append(
                        pltpu.async_copy(tab_hbm.at[ib.at[j]], rb.at[rs], gsem))
            for dsc in descs:
                dsc.wait()

            ch_base = lax.iota(jnp.int32, 16) * _SP

            def blend_body(g, c2):
                for i in range(16):
                    p = g * 16 + i
                    wx = plsc.load_gather(wxb, [jnp.zeros((16,), jnp.int32) + p])
                    wy = plsc.load_gather(wyb, [jnp.zeros((16,), jnp.int32) + p])
                    v00 = r00[p, :]
                    v01 = r01[p, :]
                    v10 = r10[p, :]
                    v11 = r11[p, :]
                    top = v00 + wx * (v01 - v00)
                    bot = v10 + wx * (v11 - v10)
                    res = top + wy * (bot - top)
                    plsc.store_scatter(obuf, [ch_base + p], res)
                return c2

            lax.fori_loop(0, _SG, blend_body, 0)

            od = [
                pltpu.async_copy(obuf.at[pl.ds(c * _SP, _SP)],
                                 out_hbm.at[b, c, yt, xt], osem)
                for c in range(_C)
            ]
            for d in od:
                d.wait()
            return carry

        lax.fori_loop(0, _SPW, strip_body, 0)

    return k(xf, table)


def kernel(x, texture):
    xf = x.reshape(2 * _NPIX)
    tex_tiles = (texture.reshape(_C, 128, 8, 8, 128)
                 .transpose(0, 1, 3, 2, 4)
                 .reshape(_C, 128, 8, 1024))
    table = _clip_transpose(tex_tiles).reshape(_TEX * _TEX, _C)
    out5 = _sc_sample(xf, table)
    return (out5.reshape(4, _C, 64, 4, 8, 128)
            .transpose(0, 1, 2, 4, 3, 5)
            .reshape(4, _C, 512, 512))


# speedup vs baseline: 2.8145x; 2.0127x over previous
import functools

import jax
import jax.numpy as jnp
from jax import lax
from jax.experimental import pallas as pl
from jax.experimental.pallas import tpu as pltpu
from jax.experimental.pallas import tpu_sc as plsc

_C = 16
_TEX = 1024
_NPIX = 4 * 512 * 512
_NW = 32
_PT = _NPIX // _NW
_N = 512
_NCH = _PT // _N
_G = _N // 16
_IPW = 512 * 512
_CLIP_LO = -123.68
_CLIP_HI = 151.061


def _clip_transpose(tex_tiles):
    mesh = plsc.VectorSubcoreMesh(core_axis_name="c", subcore_axis_name="s")
    blocks_per_w = (128 * 8) // _NW

    @functools.partial(
        pl.kernel,
        mesh=mesh,
        out_type=jax.ShapeDtypeStruct((_TEX, _TEX * _C), jnp.float32),
        compiler_params=pltpu.CompilerParams(
            needs_layout_passes=False, use_tc_tiling_on_sc=False),
        scratch_types=[
            pltpu.VMEM((_C, 1024), jnp.float32),
            pltpu.VMEM((1024 * _C,), jnp.float32),
            pltpu.SemaphoreType.DMA,
            pltpu.SemaphoreType.DMA,
        ],
    )
    def k(tex_hbm, tab_hbm, tbuf, obuf, isem, osem):
        wid = lax.axis_index("s") * 2 + lax.axis_index("c")

        def block_body(bi, carry):
            blk = wid * blocks_per_w + bi
            yt = blk // 8
            xt = blk % 8
            ind = [
                pltpu.async_copy(tex_hbm.at[c, yt, xt], tbuf.at[c], isem)
                for c in range(_C)
            ]
            for d in ind:
                d.wait()

            def tr_body(g, c2):
                t16 = g * 16 + lax.iota(jnp.int32, 16)
                for c in range(_C):
                    v = jnp.clip(tbuf[c, pl.ds(g * 16, 16)], _CLIP_LO, _CLIP_HI)
                    plsc.store_scatter(obuf, [t16 * _C + c], v)
                return c2

            lax.fori_loop(0, 64, tr_body, 0)

            outd = [
                pltpu.async_copy(
                    obuf.at[pl.ds(yin * 128 * _C, 128 * _C)],
                    tab_hbm.at[yt * 8 + yin, pl.ds(xt * 128 * _C, 128 * _C)],
                    osem)
                for yin in range(8)
            ]
            for d in outd:
                d.wait()
            return carry

        lax.fori_loop(0, blocks_per_w, block_body, 0)

    return k(tex_tiles)


_SP = 1024
_SG = _SP // 16
_SPW = (_NPIX // _SP) // _NW


def _sc_sample(xf, table):
    mesh = plsc.VectorSubcoreMesh(core_axis_name="c", subcore_axis_name="s")

    @functools.partial(
        pl.kernel,
        mesh=mesh,
        out_type=jax.ShapeDtypeStruct((4, _C, 64, 4, _SP), jnp.float32),
        compiler_params=pltpu.CompilerParams(
            needs_layout_passes=False, use_tc_tiling_on_sc=False),
        scratch_types=[
            pltpu.VMEM((8, 2, 128), jnp.float32),
            pltpu.VMEM((_SP // 128, 128), jnp.int32),
            pltpu.VMEM((_SP // 128, 128), jnp.int32),
            pltpu.VMEM((_SP // 128, 128), jnp.int32),
            pltpu.VMEM((_SP // 128, 128), jnp.int32),
            pltpu.VMEM((_SP,), jnp.float32),
            pltpu.VMEM((_SP,), jnp.float32),
            pltpu.VMEM((_SP, _C), jnp.float32),
            pltpu.VMEM((_SP, _C), jnp.float32),
            pltpu.VMEM((_SP, _C), jnp.float32),
            pltpu.VMEM((_SP, _C), jnp.float32),
            pltpu.VMEM((_C * _SP,), jnp.float32),
            pltpu.SemaphoreType.DMA,
            pltpu.SemaphoreType.DMA,
        ],
    )
    def k(x_hbm, tab_hbm, out_hbm, xbuf, i00, i01, i10, i11, wxb, wyb,
          r00, r01, r10, r11, obuf, gsem, osem):
        wid = lax.axis_index("s") * 2 + lax.axis_index("c")

        def strip_body(si, carry):
            s = wid * _SPW + si
            b = s // 256
            rem = s % 256
            yt = rem // 4
            xt = rem % 4

            xd = [
                pltpu.async_copy(x_hbm.at[b, yt * 8 + yin, xt],
                                 xbuf.at[yin], gsem)
                for yin in range(8)
            ]
            for d in xd:
                d.wait()

            def idx_body(g, c2):
                yin = g // 8
                ks = pl.ds((g % 8) * 16, 16)
                gx = xbuf[yin, 0, ks]
                gy = xbuf[yin, 1, ks]
                ix = jnp.clip((gx + 1.0) * 0.5 * (_TEX - 1), 0.0, _TEX - 1.0)
                iy = jnp.clip((gy + 1.0) * 0.5 * (_TEX - 1), 0.0, _TEX - 1.0)
                x0 = ix.astype(jnp.int32)
                y0 = iy.astype(jnp.int32)
                wx = ix - x0.astype(jnp.float32)
                wy = iy - y0.astype(jnp.float32)
                x1 = jnp.minimum(x0 + 1, _TEX - 1)
                y1 = jnp.minimum(y0 + 1, _TEX - 1)
                row0 = y0 * _TEX
                row1 = y1 * _TEX
                j = g // 8
                sl = pl.ds((g % 8) * 16, 16)
                i00[j, sl] = row0 + x0
                i01[j, sl] = row0 + x1
                i10[j, sl] = row1 + x0
                i11[j, sl] = row1 + x1
                fl = pl.ds(g * 16, 16)
                wxb[fl] = wx
                wyb[fl] = wy
                return c2

            lax.fori_loop(0, _SG, idx_body, 0)

            descs = []
            for j in range(_SP // 128):
                rs = pl.ds(j * 128, 128)
                for ib, rb in ((i00, r00), (i01, r01), (i10, r10), (i11, r11)):
                    descs.append(
                        pltpu.async_copy(tab_hbm.at[ib.at[j]], rb.at[rs], gsem))
            for dsc in descs:
                dsc.wait()

            ch_base = lax.iota(jnp.int32, 16) * _SP

            def blend_body(g, c2):
                for i in range(16):
                    p = g * 16 + i
                    wx = plsc.load_gather(wxb, [jnp.zeros((16,), jnp.int32) + p])
                    wy = plsc.load_gather(wyb, [jnp.zeros((16,), jnp.int32) + p])
                    v00 = r00[p, :]
                    v01 = r01[p, :]
                    v10 = r10[p, :]
                    v11 = r11[p, :]
                    top = v00 + wx * (v01 - v00)
                    bot = v10 + wx * (v11 - v10)
                    res = top + wy * (bot - top)
                    plsc.store_scatter(obuf, [ch_base + p], res)
                return c2

            lax.fori_loop(0, _SG, blend_body, 0)

            od = [
                pltpu.async_copy(obuf.at[pl.ds(c * _SP, _SP)],
                                 out_hbm.at[b, c, yt, xt], osem)
                for c in range(_C)
            ]
            for d in od:
                d.wait()
            return carry

        lax.fori_loop(0, _SPW, strip_body, 0)

    return k(xf, table)


def kernel(x, texture):
    xf = x.reshape(4, 512, 4, 128, 2).transpose(0, 1, 2, 4, 3)
    tex_tiles = (texture.reshape(_C, 128, 8, 8, 128)
                 .transpose(0, 1, 3, 2, 4)
                 .reshape(_C, 128, 8, 1024))
    table = _clip_transpose(tex_tiles).reshape(_TEX * _TEX, _C)
    out5 = _sc_sample(xf, table)
    return (out5.reshape(4, _C, 64, 4, 8, 128)
            .transpose(0, 1, 2, 4, 3, 5)
            .reshape(4, _C, 512, 512))


# speedup vs baseline: 3.2081x; 1.1399x over previous
import functools

import jax
import jax.numpy as jnp
from jax import lax
from jax.experimental import pallas as pl
from jax.experimental.pallas import tpu as pltpu
from jax.experimental.pallas import tpu_sc as plsc

_C = 16
_TEX = 1024
_NPIX = 4 * 512 * 512
_NW = 32
_IPW = 512 * 512
_SP = 1024
_SPW = (_NPIX // _SP) // _NW
_CLIP_LO = -123.68
_CLIP_HI = 151.061

_SC_PARAMS = pltpu.CompilerParams(
    needs_layout_passes=False, use_tc_tiling_on_sc=False)


def _clip_transpose(tex_tiles):
    mesh = plsc.VectorSubcoreMesh(core_axis_name="c", subcore_axis_name="s")

    @functools.partial(
        pl.kernel,
        mesh=mesh,
        out_type=jax.ShapeDtypeStruct((_TEX, _TEX * _C), jnp.float32),
        compiler_params=_SC_PARAMS,
        scratch_types=[
            pltpu.VMEM((_C, 1024), jnp.float32),
            pltpu.VMEM((_C, 1024), jnp.float32),
            pltpu.VMEM((1024 * _C,), jnp.float32),
            pltpu.VMEM((1024 * _C,), jnp.float32),
            pltpu.SemaphoreType.DMA,
            pltpu.SemaphoreType.DMA,
            pltpu.SemaphoreType.DMA,
            pltpu.SemaphoreType.DMA,
        ],
    )
    def k(tex_hbm, tab_hbm, tba, tbb, oba, obb, isa, isb, osa, osb):
        wid = lax.axis_index("s") * 2 + lax.axis_index("c")
        i16 = lax.iota(jnp.int32, 16)

        def tin(blk, tb, sem, fire):
            yt = blk // 8
            xt = blk % 8
            for c in range(_C):
                d = (pltpu.async_copy if fire else pltpu.make_async_copy)(
                    tex_hbm.at[c, yt, xt], tb.at[c], sem)
                if not fire:
                    d.wait()

        def tout(blk, ob, sem, fire):
            yt = blk // 8
            xt = blk % 8
            for yin in range(8):
                d = (pltpu.async_copy if fire else pltpu.make_async_copy)(
                    ob.at[pl.ds(yin * 2048, 2048)],
                    tab_hbm.at[yt * 8 + yin, pl.ds(xt * 2048, 2048)], sem)
                if not fire:
                    d.wait()

        def transpose_block(tb, ob):
            def tr(g, c2):
                t16 = (g * 16 + i16) * _C
                for c in range(_C):
                    v = jnp.clip(tb[c, pl.ds(g * 16, 16)], _CLIP_LO, _CLIP_HI)
                    plsc.store_scatter(ob, [t16 + c], v)
                return c2

            lax.fori_loop(0, 64, tr, 0)

        def half_iter(blk, tb, ob, isem, osem, it):
            @pl.when(it > 0)
            def _():
                tout(blk, ob, osem, False)

            transpose_block(tb, ob)
            tout(blk, ob, osem, True)
            @pl.when(it < 15)
            def _():
                tin(blk + 2, tb, isem, True)

        def pair(it, carry):
            blk_a = wid * 32 + 2 * it
            tin(blk_a, tba, isa, False)
            half_iter(blk_a, tba, oba, isa, osa, it)
            tin(blk_a + 1, tbb, isb, False)
            half_iter(blk_a + 1, tbb, obb, isb, osb, it)
            return carry

        tin(wid * 32, tba, isa, True)
        tin(wid * 32 + 1, tbb, isb, True)
        lax.fori_loop(0, 16, pair, 0)
        tout(0, oba, osa, False)
        tout(0, obb, osb, False)

    return k(tex_tiles)


def _sc_sample(xf, table):
    mesh = plsc.VectorSubcoreMesh(core_axis_name="c", subcore_axis_name="s")

    @functools.partial(
        pl.kernel,
        mesh=mesh,
        out_type=jax.ShapeDtypeStruct((4, _C, 64, 4, _SP), jnp.float32),
        compiler_params=_SC_PARAMS,
        scratch_types=[
            pltpu.VMEM((8, 2, 128), jnp.float32),
            pltpu.VMEM((8, 2, 128), jnp.float32),
            pltpu.VMEM((8, 128), jnp.int32),
            pltpu.VMEM((8, 128), jnp.int32),
            pltpu.VMEM((8, 128), jnp.int32),
            pltpu.VMEM((8, 128), jnp.int32),
            pltpu.VMEM((_SP,), jnp.float32),
            pltpu.VMEM((_SP,), jnp.float32),
        ] + [pltpu.VMEM((_SP // 2, _C), jnp.float32)] * 8 + [
            pltpu.VMEM((_C * _SP,), jnp.float32),
            pltpu.VMEM((_C * _SP,), jnp.float32),
            pltpu.SemaphoreType.DMA,
            pltpu.SemaphoreType.DMA,
            pltpu.SemaphoreType.DMA,
            pltpu.SemaphoreType.DMA,
            pltpu.SemaphoreType.DMA,
            pltpu.SemaphoreType.DMA,
        ],
    )
    def k(x_hbm, tab_hbm, out_hbm, xba, xbb, i00, i01, i10, i11, wxb, wyb,
          r00a, r00b, r01a, r01b, r10a, r10b, r11a, r11b,
          oba, obb, xsa, xsb, gs0, gs1, osa, osb):
        wid = lax.axis_index("s") * 2 + lax.axis_index("c")
        i16 = lax.iota(jnp.int32, 16)
        ch_base = i16 * _SP
        idx = (i00, i01, i10, i11)
        rows = ((r00a, r00b), (r01a, r01b), (r10a, r10b), (r11a, r11b))

        def coords(s):
            b = s // 256
            rem = s % 256
            return b, rem // 4, rem % 4

        def xmove(s, xb, sem, fire):
            b, yt, xt = coords(s)
            for yin in range(8):
                d = (pltpu.async_copy if fire else pltpu.make_async_copy)(
                    x_hbm.at[b, yt * 8 + yin, xt], xb.at[yin], sem)
                if not fire:
                    d.wait()

        def idx_compute(xb):
            def idx_body(g, c2):
                yin = g // 8
                ks = pl.ds((g % 8) * 16, 16)
                gx = xb[yin, 0, ks]
                gy = xb[yin, 1, ks]
                ix = jnp.clip((gx + 1.0) * 0.5 * (_TEX - 1), 0.0, _TEX - 1.0)
                iy = jnp.clip((gy + 1.0) * 0.5 * (_TEX - 1), 0.0, _TEX - 1.0)
                x0 = ix.astype(jnp.int32)
                y0 = iy.astype(jnp.int32)
                wx = ix - x0.astype(jnp.float32)
                wy = iy - y0.astype(jnp.float32)
                x1 = jnp.minimum(x0 + 1, _TEX - 1)
                y1 = jnp.minimum(y0 + 1, _TEX - 1)
                row0 = y0 * _TEX
                row1 = y1 * _TEX
                j = g // 8
                sl = pl.ds((g % 8) * 16, 16)
                i00[j, sl] = row0 + x0
                i01[j, sl] = row0 + x1
                i10[j, sl] = row1 + x0
                i11[j, sl] = row1 + x1
                fl = pl.ds(g * 16, 16)
                wxb[fl] = wx
                wyb[fl] = wy
                return c2

            lax.fori_loop(0, 64, idx_body, 0)

        def gathers(half, sem, fire):
            for jj in range(4):
                j = half * 4 + jj
                rs = pl.ds(jj * 128, 128)
                for nb in range(4):
                    d = (pltpu.async_copy if fire else pltpu.make_async_copy)(
                        tab_hbm.at[idx[nb].at[j]], rows[nb][half].at[rs], sem)
                    if not fire:
                        d.wait()

        def blend(half, ob):
            def blend_body(g2, c2):
                g = half * 32 + g2
                for i in range(16):
                    p = g * 16 + i
                    ph = g2 * 16 + i
                    zp = jnp.zeros((16,), jnp.int32) + p
                    wx = plsc.load_gather(wxb, [zp])
                    wy = plsc.load_gather(wyb, [zp])
                    v00 = rows[0][half][ph, :]
                    v01 = rows[1][half][ph, :]
                    v10 = rows[2][half][ph, :]
                    v11 = rows[3][half][ph, :]
                    top = v00 + wx * (v01 - v00)
                    bot = v10 + wx * (v11 - v10)
                    res = top + wy * (bot - top)
                    plsc.store_scatter(ob, [ch_base + p], res)
                return c2

            lax.fori_loop(0, 32, blend_body, 0)

        def omove(s, ob, sem, fire):
            b, yt, xt = coords(s)
            for c in range(_C):
                d = (pltpu.async_copy if fire else pltpu.make_async_copy)(
                    ob.at[pl.ds(c * _SP, _SP)], out_hbm.at[b, c, yt, xt], sem)
                if not fire:
                    d.wait()

        _SERIAL = False

        def do_strip(s, xb, ob, osem, it):
            idx_compute(xb)
            if _SERIAL:
                gathers(0, gs0, True)
                gathers(0, gs0, False)
                gathers(1, gs1, True)
                gathers(1, gs1, False)
                blend(0, ob)
                blend(1, ob)
                omove(s, ob, osem, True)
                omove(s, ob, osem, False)
                return
            gathers(0, gs0, True)
            gathers(1, gs1, True)
            @pl.when(it > 0)
            def _():
                omove(s, ob, osem, False)

            gathers(0, gs0, False)
            blend(0, ob)
            gathers(1, gs1, False)
            blend(1, ob)
            omove(s, ob, osem, True)

        def pair(it, carry):
            s_a = wid * _SPW + 2 * it
            xmove(s_a + 1, xbb, xsb, True)
            xmove(s_a, xba, xsa, False)
            do_strip(s_a, xba, oba, osa, it)
            @pl.when(it < _SPW // 2 - 1)
            def _():
                xmove(s_a + 2, xba, xsa, True)

            xmove(s_a + 1, xbb, xsb, False)
            do_strip(s_a + 1, xbb, obb, osb, it)
            return carry

        xmove(wid * _SPW, xba, xsa, True)
        lax.fori_loop(0, _SPW // 2, pair, 0)
        if not _SERIAL:
            omove(0, oba, osa, False)
            omove(0, obb, osb, False)

    return k(xf, table)


def kernel(x, texture):
    xf = x.reshape(4, 512, 4, 128, 2).transpose(0, 1, 2, 4, 3)
    tex_tiles = (texture.reshape(_C, 128, 8, 8, 128)
                 .transpose(0, 1, 3, 2, 4)
                 .reshape(_C, 128, 8, 1024))
    table = _clip_transpose(tex_tiles).reshape(_TEX * _TEX, _C)
    out5 = _sc_sample(xf, table)
    return (out5.reshape(4, _C, 64, 4, 8, 128)
            .transpose(0, 1, 2, 4, 3, 5)
            .reshape(4, _C, 512, 512))
